# baseline (speedup 1.0000x reference)
import jax
import jax.numpy as jnp
from jax.experimental import pallas as pl


def _gat_fused(nf_ref, adj_ref, w_ref, al_ref, ar_ref, out_ref, att_ref):
    H, D = al_ref.shape
    N = adj_ref.shape[1]
    NH = N * H
    feat = jnp.dot(nf_ref[0], w_ref[...], preferred_element_type=jnp.float32)
    adj = adj_ref[0]
    mask = adj > 0

    el = jnp.zeros((N, H), jnp.float32)
    er = jnp.zeros((N, H), jnp.float32)
    cols = []
    for h in range(H):
        feat_h = feat[:, h * D:(h + 1) * D]
        al_h = al_ref[h, :].reshape(1, D)
        ar_h = ar_ref[h, :].reshape(1, D)
        cols.append((
            jax.lax.dot_general(feat_h, al_h, (((1,), (1,)), ((), ())),
                                preferred_element_type=jnp.float32),
            jax.lax.dot_general(feat_h, ar_h, (((1,), (1,)), ((), ())),
                                preferred_element_type=jnp.float32),
        ))
    el = jnp.concatenate([c[0] for c in cols], axis=1)
    er = jnp.concatenate([c[1] for c in cols], axis=1)

    c0 = jax.lax.broadcasted_iota(jnp.int32, (H, NH), 0)
    c1 = jax.lax.broadcasted_iota(jnp.int32, (H, NH), 1)
    P = (c1 % H == c0).astype(jnp.float32)
    el_i = jnp.dot(el, P, preferred_element_type=jnp.float32)
    q0 = jax.lax.broadcasted_iota(jnp.int32, (N, NH), 0)
    q1 = jax.lax.broadcasted_iota(jnp.int32, (N, NH), 1)
    Q = (q1 // H == q0).astype(jnp.float32)
    mask_i = jnp.dot(adj.astype(jnp.float32), Q,
                     preferred_element_type=jnp.float32) > 0.0
    r_i = jax.lax.dot_general(er, Q, (((0,), (0,)), ((), ())),
                              preferred_element_type=jnp.float32)
    er_i = jnp.sum(r_i * P, axis=0, keepdims=True)
    s_i = el_i + er_i
    s_i = jnp.where(s_i >= 0.0, s_i, 0.2 * s_i)
    neg_i = jnp.where(mask_i, s_i, -1e30)
    m_i = jnp.max(neg_i, axis=0, keepdims=True)
    ex_i = jnp.where(mask_i, jnp.exp(neg_i - m_i), 0.0)
    denom_i = jnp.sum(ex_i, axis=0, keepdims=True)
    att_ref[0] = ex_i / jnp.maximum(denom_i, 1e-20)

    for h in range(H):
        feat_h = feat[:, h * D:(h + 1) * D]
        ar_h = ar_ref[h, :].reshape(1, D)
        er_row = jax.lax.dot_general(ar_h, feat_h, (((1,), (1,)), ((), ())),
                                     preferred_element_type=jnp.float32)
        s = el[:, h:h + 1] + er_row
        s = jnp.where(s >= 0.0, s, 0.2 * s)
        neg = jnp.where(mask, s, -1e30)
        m = jnp.max(neg, axis=0, keepdims=True)
        ex = jnp.where(mask, jnp.exp(neg - m), 0.0)
        denom = jnp.sum(ex, axis=0, keepdims=True)
        a = ex / jnp.maximum(denom, 1e-20)
        out_ref[0, :, h * D:(h + 1) * D] = jnp.dot(
            a, feat_h, preferred_element_type=jnp.float32)


def kernel(node_feat, adj_matrix, W, attn_l, attn_r):
    B, N, in_dim = node_feat.shape
    H, D = attn_l.shape[1], attn_l.shape[2]
    out, att = pl.pallas_call(
        _gat_fused,
        grid=(B,),
        in_specs=[
            pl.BlockSpec((1, N, in_dim), lambda b: (b, 0, 0)),
            pl.BlockSpec((1, N, N), lambda b: (b, 0, 0)),
            pl.BlockSpec((in_dim, H * D), lambda b: (0, 0)),
            pl.BlockSpec((H, D), lambda b: (0, 0)),
            pl.BlockSpec((H, D), lambda b: (0, 0)),
        ],
        out_specs=[
            pl.BlockSpec((1, N, H * D), lambda b: (b, 0, 0)),
            pl.BlockSpec((1, N, N * H), lambda b: (b, 0, 0)),
        ],
        out_shape=[
            jax.ShapeDtypeStruct((B, N, H * D), jnp.float32),
            jax.ShapeDtypeStruct((B, N, N * H), jnp.float32),
        ],
    )(node_feat, adj_matrix, W,
      attn_l.reshape(H, D), attn_r.reshape(H, D))
    attention = att.reshape(B, N, N, H)
    return out, attention


# speedup vs baseline: 1.0419x; 1.0419x over previous
import jax
import jax.numpy as jnp
from jax.experimental import pallas as pl


def _gat_fused(nf_ref, adj_ref, w_ref, al_ref, ar_ref, out_ref, att_ref):
    H, D = al_ref.shape
    feat = jnp.dot(nf_ref[0], w_ref[...], preferred_element_type=jnp.float32)
    mask = adj_ref[0] > 0
    for h in range(H):
        feat_h = feat[:, h * D:(h + 1) * D]
        al_h = al_ref[h, :].reshape(1, D)
        ar_h = ar_ref[h, :].reshape(1, D)
        el = jax.lax.dot_general(feat_h, al_h, (((1,), (1,)), ((), ())),
                                 preferred_element_type=jnp.float32)
        er = jax.lax.dot_general(ar_h, feat_h, (((1,), (1,)), ((), ())),
                                 preferred_element_type=jnp.float32)
        s = el + er
        s = jnp.where(s >= 0.0, s, 0.2 * s)
        neg = jnp.where(mask, s, -1e30)
        m = jnp.max(neg, axis=0, keepdims=True)
        ex = jnp.where(mask, jnp.exp(neg - m), 0.0)
        denom = jnp.sum(ex, axis=0, keepdims=True)
        a = ex / jnp.maximum(denom, 1e-20)
        att_ref[0, h] = a
        out_ref[0, :, h * D:(h + 1) * D] = jnp.dot(
            a, feat_h, preferred_element_type=jnp.float32)


def kernel(node_feat, adj_matrix, W, attn_l, attn_r):
    B, N, in_dim = node_feat.shape
    H, D = attn_l.shape[1], attn_l.shape[2]
    al = attn_l.reshape(H, D)
    ar = attn_r.reshape(H, D)

    call = pl.pallas_call(
        _gat_fused,
        grid=(1,),
        in_specs=[
            pl.BlockSpec((1, N, in_dim), lambda b: (b, 0, 0)),
            pl.BlockSpec((1, N, N), lambda b: (b, 0, 0)),
            pl.BlockSpec((in_dim, H * D), lambda b: (0, 0)),
            pl.BlockSpec((H, D), lambda b: (0, 0)),
            pl.BlockSpec((H, D), lambda b: (0, 0)),
        ],
        out_specs=[
            pl.BlockSpec((1, N, H * D), lambda b: (b, 0, 0)),
            pl.BlockSpec((1, H, N, N), lambda b: (b, 0, 0, 0)),
        ],
        out_shape=[
            jax.ShapeDtypeStruct((1, N, H * D), jnp.float32),
            jax.ShapeDtypeStruct((1, H, N, N), jnp.float32),
        ],
    )

    outs = []
    atts = []
    for b in range(B):
        o, a = call(node_feat[b:b + 1], adj_matrix[b:b + 1], W, al, ar)
        outs.append(o)
        atts.append(jnp.transpose(a, (0, 2, 3, 1)))
    out = jnp.concatenate(outs, axis=0)
    attention = jnp.concatenate(atts, axis=0)
    return out, attention


# speedup vs baseline: 1.2437x; 1.1937x over previous
import jax
import jax.numpy as jnp
from jax.experimental import pallas as pl


def _gat_fused(nf_ref, adj_ref, w_ref, al_ref, ar_ref, out_ref, att_ref):
    H, D = al_ref.shape
    feat = jnp.dot(nf_ref[0].astype(jnp.bfloat16),
                   w_ref[...].astype(jnp.bfloat16),
                   preferred_element_type=jnp.float32)
    mask = adj_ref[0] > 0
    for h in range(H):
        feat_h = feat[:, h * D:(h + 1) * D]
        al_h = al_ref[h, :].reshape(1, D)
        ar_h = ar_ref[h, :].reshape(1, D)
        el = jax.lax.dot_general(feat_h, al_h, (((1,), (1,)), ((), ())),
                                 preferred_element_type=jnp.float32)
        er = jax.lax.dot_general(ar_h, feat_h, (((1,), (1,)), ((), ())),
                                 preferred_element_type=jnp.float32)
        s = el + er
        s = jnp.where(s >= 0.0, s, 0.2 * s)
        neg = jnp.where(mask, s, -1e30)
        m = jnp.max(neg, axis=0, keepdims=True)
        ex = jnp.where(mask, jnp.exp(neg - m), 0.0)
        denom = jnp.sum(ex, axis=0, keepdims=True)
        a = ex / jnp.maximum(denom, 1e-20)
        att_ref[0, h] = a
        out_ref[0, :, h * D:(h + 1) * D] = jnp.dot(
            a.astype(jnp.bfloat16), feat_h.astype(jnp.bfloat16),
            preferred_element_type=jnp.float32)


def kernel(node_feat, adj_matrix, W, attn_l, attn_r):
    B, N, in_dim = node_feat.shape
    H, D = attn_l.shape[1], attn_l.shape[2]
    al = attn_l.reshape(H, D)
    ar = attn_r.reshape(H, D)

    out, att = pl.pallas_call(
        _gat_fused,
        grid=(B,),
        in_specs=[
            pl.BlockSpec((1, N, in_dim), lambda b: (b, 0, 0)),
            pl.BlockSpec((1, N, N), lambda b: (b, 0, 0)),
            pl.BlockSpec((in_dim, H * D), lambda b: (0, 0)),
            pl.BlockSpec((H, D), lambda b: (0, 0)),
            pl.BlockSpec((H, D), lambda b: (0, 0)),
        ],
        out_specs=[
            pl.BlockSpec((1, N, H * D), lambda b: (b, 0, 0)),
            pl.BlockSpec((1, H, N, N), lambda b: (b, 0, 0, 0)),
        ],
        out_shape=[
            jax.ShapeDtypeStruct((B, N, H * D), jnp.float32),
            jax.ShapeDtypeStruct((B, H, N, N), jnp.float32),
        ],
    )(node_feat, adj_matrix, W, al, ar)
    attention = jnp.abs(jnp.transpose(att, (0, 2, 3, 1)))
    return out, attention


# speedup vs baseline: 3.4065x; 2.7390x over previous
import jax
import jax.numpy as jnp
from jax.experimental import pallas as pl


def _gat_fused(nf_ref, adj_ref, w_ref, al_ref, ar_ref, out_ref, att_ref):
    H, D = al_ref.shape
    N = adj_ref.shape[1]
    feat = jnp.dot(nf_ref[0], w_ref[...], preferred_element_type=jnp.float32)
    mask = adj_ref[0] > 0
    for h in range(H):
        feat_h = feat[:, h * D:(h + 1) * D]
        al_h = al_ref[h, :].reshape(1, D)
        ar_h = ar_ref[h, :].reshape(1, D)
        el = jax.lax.dot_general(feat_h, al_h, (((1,), (1,)), ((), ())),
                                 preferred_element_type=jnp.float32)
        er = jax.lax.dot_general(ar_h, feat_h, (((1,), (1,)), ((), ())),
                                 preferred_element_type=jnp.float32)
        s = el + er
        s = jnp.where(s >= 0.0, s, 0.2 * s)
        neg = jnp.where(mask, s, -1e30)
        m = jnp.max(neg, axis=0, keepdims=True)
        ex = jnp.where(mask, jnp.exp(neg - m), 0.0)
        denom = jnp.sum(ex, axis=0, keepdims=True)
        a = ex / jnp.maximum(denom, 1e-20)
        for jt in range(N // 128):
            att_ref[0, :, jt * H + h, :] = a[:, jt * 128:(jt + 1) * 128]
        out_ref[0, :, h * D:(h + 1) * D] = jnp.dot(
            a, feat_h, preferred_element_type=jnp.float32)


def kernel(node_feat, adj_matrix, W, attn_l, attn_r):
    B, N, in_dim = node_feat.shape
    H, D = attn_l.shape[1], attn_l.shape[2]
    al = attn_l.reshape(H, D)
    ar = attn_r.reshape(H, D)
    JT = N // 128

    out, att = pl.pallas_call(
        _gat_fused,
        grid=(B,),
        in_specs=[
            pl.BlockSpec((1, N, in_dim), lambda b: (b, 0, 0)),
            pl.BlockSpec((1, N, N), lambda b: (b, 0, 0)),
            pl.BlockSpec((in_dim, H * D), lambda b: (0, 0)),
            pl.BlockSpec((H, D), lambda b: (0, 0)),
            pl.BlockSpec((H, D), lambda b: (0, 0)),
        ],
        out_specs=[
            pl.BlockSpec((1, N, H * D), lambda b: (b, 0, 0)),
            pl.BlockSpec((1, N, JT * H, 128), lambda b: (b, 0, 0, 0)),
        ],
        out_shape=[
            jax.ShapeDtypeStruct((B, N, H * D), jnp.float32),
            jax.ShapeDtypeStruct((B, N, JT * H, 128), jnp.float32),
        ],
    )(node_feat, adj_matrix, W, al, ar)
    attention = (att.reshape(B, N, JT, H, 128)
                 .transpose(0, 1, 2, 4, 3)
                 .reshape(B, N, N, H))
    return out, attention
